# Initial kernel scaffold; baseline (speedup 1.0000x reference)
import jax
import jax.numpy as jnp
from jax.experimental import pallas as pl


def kernel(t, pe):
    raise NotImplementedError("write your pallas kernel here")


# speedup vs baseline: 1.8460x; 1.8460x over previous
import functools

import jax
import jax.numpy as jnp
from jax import lax
from jax.experimental import pallas as pl
from jax.experimental.pallas import tpu as pltpu
from jax.experimental.pallas import tpu_sc as plsc

DIM = 4096
NUM_CORES = 2
NUM_SUBCORES = 16
NUM_WORKERS = NUM_CORES * NUM_SUBCORES
CHUNK = 8


@functools.partial(jax.jit, static_argnums=(2, 3))
def _gather_sc(idx, pe, b_per_w, n_chunks):
    mesh = plsc.VectorSubcoreMesh(
        core_axis_name="c", subcore_axis_name="s", num_cores=NUM_CORES
    )

    @functools.partial(
        pl.kernel,
        out_type=jax.ShapeDtypeStruct((NUM_WORKERS * b_per_w, DIM), jnp.float32),
        mesh=mesh,
        scratch_types=[
            pltpu.VMEM((n_chunks, CHUNK), jnp.int32),
            pltpu.VMEM((CHUNK, DIM), jnp.float32),
            pltpu.VMEM((CHUNK, DIM), jnp.float32),
            pltpu.SemaphoreType.DMA,
            pltpu.SemaphoreType.DMA,
            pltpu.SemaphoreType.DMA,
            pltpu.SemaphoreType.DMA,
        ],
    )
    def k(idx_hbm, table_hbm, out_hbm, idx_v, buf0, buf1, gsem0, gsem1, ssem0, ssem1):
        wid = lax.axis_index("s") * NUM_CORES + lax.axis_index("c")
        base = wid * b_per_w

        pltpu.sync_copy(idx_hbm.at[wid], idx_v)

        def gather(j, buf, sem):
            return pltpu.async_copy(table_hbm.at[idx_v.at[j]], buf, sem)

        def scatter(j, buf, sem):
            return pltpu.async_copy(buf, out_hbm.at[pl.ds(base + j * CHUNK, CHUNK)], sem)

        def wait_scatter(buf, sem):
            pltpu.make_async_copy(buf, out_hbm.at[pl.ds(base, CHUNK)], sem).wait()

        gather(0, buf0, gsem0)

        def body(i, carry):
            j0 = i * 2
            pltpu.make_async_copy(table_hbm.at[idx_v.at[j0]], buf0, gsem0).wait()

            @pl.when(j0 > 0)
            def _():
                wait_scatter(buf1, ssem1)

            gather(j0 + 1, buf1, gsem1)
            scatter(j0, buf0, ssem0)

            pltpu.make_async_copy(table_hbm.at[idx_v.at[j0 + 1]], buf1, gsem1).wait()

            @pl.when(j0 < n_chunks - 2)
            def _():
                wait_scatter(buf0, ssem0)
                gather(j0 + 2, buf0, gsem0)

            scatter(j0 + 1, buf1, ssem1)
            return carry

        lax.fori_loop(0, n_chunks // 2, body, 0)

        wait_scatter(buf0, ssem0)
        wait_scatter(buf1, ssem1)

    return k(idx, pe)


def kernel(t, pe):
    batch, seq = t.shape
    total = batch * seq
    b_per_w = total // NUM_WORKERS
    n_chunks = b_per_w // CHUNK
    idx = t.astype(jnp.int32).reshape(NUM_WORKERS, n_chunks, CHUNK)
    out = _gather_sc(idx, pe, b_per_w, n_chunks)
    return out.reshape(batch, seq, DIM)


# speedup vs baseline: 1.8678x; 1.0118x over previous
import functools

import jax
import jax.numpy as jnp
from jax import lax
from jax.experimental import pallas as pl
from jax.experimental.pallas import tpu as pltpu
from jax.experimental.pallas import tpu_sc as plsc

DIM = 4096
NUM_CORES = 2
NUM_SUBCORES = 16
NUM_WORKERS = NUM_CORES * NUM_SUBCORES
CHUNK = 8
NBUF = 3


@functools.partial(jax.jit, static_argnums=(2, 3))
def _gather_sc(idx, pe, b_per_w, n_chunks):
    mesh = plsc.VectorSubcoreMesh(
        core_axis_name="c", subcore_axis_name="s", num_cores=NUM_CORES
    )
    n_main = (n_chunks // NBUF) * NBUF if n_chunks % NBUF else n_chunks - NBUF
    while n_chunks - n_main < NBUF - 1:
        n_main -= NBUF

    @functools.partial(
        pl.kernel,
        out_type=jax.ShapeDtypeStruct((NUM_WORKERS * b_per_w, DIM), jnp.float32),
        mesh=mesh,
        scratch_types=[
            pltpu.VMEM((n_chunks, CHUNK), jnp.int32),
            *([pltpu.VMEM((CHUNK, DIM), jnp.float32)] * NBUF),
            *([pltpu.SemaphoreType.DMA] * (2 * NBUF)),
        ],
    )
    def k(idx_hbm, table_hbm, out_hbm, idx_v, *bufs_and_sems):
        bufs = bufs_and_sems[:NBUF]
        gsem = bufs_and_sems[NBUF : 2 * NBUF]
        ssem = bufs_and_sems[2 * NBUF :]

        wid = lax.axis_index("s") * NUM_CORES + lax.axis_index("c")
        base = wid * b_per_w

        pltpu.sync_copy(idx_hbm.at[wid], idx_v)

        def start_gather(j, b):
            pltpu.async_copy(table_hbm.at[idx_v.at[j]], bufs[b], gsem[b])

        def wait_gather(j, b):
            pltpu.make_async_copy(table_hbm.at[idx_v.at[j]], bufs[b], gsem[b]).wait()

        def start_scatter(j, b):
            pltpu.async_copy(bufs[b], out_hbm.at[pl.ds(base + j * CHUNK, CHUNK)], ssem[b])

        def wait_scatter(b):
            pltpu.make_async_copy(bufs[b], out_hbm.at[pl.ds(base, CHUNK)], ssem[b]).wait()

        start_gather(0, 0)
        start_gather(1, 1)

        def body(i, carry):
            j0 = i * NBUF
            for kk in range(NBUF):
                j = j0 + kk
                b = kk
                pf = (kk + 2) % NBUF
                wait_gather(j, b)
                start_scatter(j, b)
                if kk == 0:
                    @pl.when(i > 0)
                    def _():
                        wait_scatter(pf)
                else:
                    wait_scatter(pf)
                start_gather(j + 2, pf)
            return carry

        lax.fori_loop(0, n_main // NBUF, body, 0)

        for j in range(n_main, n_chunks):
            b = j % NBUF
            if j >= n_main + 2:
                wait_scatter(b)
                start_gather(j, b)
            wait_gather(j, b)
            start_scatter(j, b)

        for j in range(n_chunks - NBUF, n_chunks):
            wait_scatter(j % NBUF)

    return k(idx, pe)


def kernel(t, pe):
    batch, seq = t.shape
    total = batch * seq
    b_per_w = total // NUM_WORKERS
    n_chunks = b_per_w // CHUNK
    idx = t.astype(jnp.int32).reshape(NUM_WORKERS, n_chunks, CHUNK)
    out = _gather_sc(idx, pe, b_per_w, n_chunks)
    return out.reshape(batch, seq, DIM)
